# baseline (speedup 1.0000x reference)
import functools

import jax
import jax.numpy as jnp
from jax import lax
from jax.experimental import pallas as pl
from jax.experimental.pallas import tpu as pltpu
from jax.experimental.pallas import tpu_sc as plsc

NC = 2
NS = 16
NW = NC * NS
GR = 16


def _sc_scores(B, K, D, center, outside, negflat, emb_v, emb_u):
    RPW = B // NW
    NG = RPW // GR
    NIDX = RPW * K
    GROWS = GR * (K + 2)
    NGR = GR * K
    assert NG % 2 == 0

    mesh = plsc.VectorSubcoreMesh(core_axis_name="c", subcore_axis_name="s")
    f32 = jnp.float32

    def body(center_r, outside_r, negative_r, ev, eu, scores_hbm,
             idx_c, idx_o, idx_n, buf, scores, gsem):
        wid = lax.axis_index("s") * NC + lax.axis_index("c")

        pltpu.sync_copy(center_r.at[pl.ds(wid * RPW, RPW)], idx_c)
        pltpu.sync_copy(outside_r.at[pl.ds(wid * RPW, RPW)], idx_o)
        pltpu.sync_copy(negative_r.at[pl.ds(wid * NIDX, NIDX)], idx_n)

        def xfers(s, g):
            so = s * GROWS
            res = [(ev, idx_c.at[pl.ds(g * GR, GR)], buf.at[pl.ds(so, GR)]),
                   (eu, idx_o.at[pl.ds(g * GR, GR)],
                    buf.at[pl.ds(so + GR, GR)])]
            off = 0
            while off < NGR:
                n = min(128, NGR - off)
                res.append((eu, idx_n.at[pl.ds(g * NGR + off, n)],
                            buf.at[pl.ds(so + 2 * GR + off, n)]))
                off += n
            return res

        def fire(s, g):
            for tab, idx, dst in xfers(s, g):
                pltpu.async_copy(tab.at[idx], dst, gsem)

        def drain(s, g):
            for tab, idx, dst in xfers(s, g):
                pltpu.make_async_copy(tab.at[idx], dst, gsem).wait()

        iota = lax.iota(jnp.int32, 16)

        def compute(s, g):
            so = s * GROWS
            vrows = so + iota
            orows = vrows + GR
            nrows = so + 2 * GR + iota * K

            def jstep(j, carry):
                col = jnp.full((16,), j, jnp.int32)
                vcol = plsc.load_gather(buf, [vrows, col])
                ocol = plsc.load_gather(buf, [orows, col])
                out = [carry[0] + vcol * ocol]
                for k in range(K):
                    ncol = plsc.load_gather(buf, [nrows + k, col])
                    out.append(carry[1 + k] + ncol * vcol)
                return tuple(out)

            zero = jnp.zeros((16,), f32)
            accs = lax.fori_loop(0, D, jstep, (zero,) * (K + 1))
            for k in range(K + 1):
                scores[k, pl.ds(g * GR, GR)] = accs[k]

        fire(0, 0)

        def pair(i, carry):
            g0 = 2 * i
            drain(0, g0)
            fire(1, g0 + 1)
            compute(0, g0)
            g1 = 2 * i + 1
            drain(1, g1)
            fire(0, g1 + 1)
            compute(1, g1)
            return carry

        lax.fori_loop(0, NG // 2 - 1, pair, 0)

        g0 = NG - 2
        drain(0, g0)
        fire(1, g0 + 1)
        compute(0, g0)
        drain(1, g0 + 1)
        compute(1, g0 + 1)

        pltpu.sync_copy(scores, scores_hbm.at[:, pl.ds(wid * RPW, RPW)])

    run = pl.kernel(
        body,
        out_type=jax.ShapeDtypeStruct((K + 1, B), f32),
        mesh=mesh,
        compiler_params=pltpu.CompilerParams(
            use_tc_tiling_on_sc=False, needs_layout_passes=False),
        scratch_types=[
            pltpu.VMEM((RPW,), jnp.int32),
            pltpu.VMEM((RPW,), jnp.int32),
            pltpu.VMEM((NIDX,), jnp.int32),
            pltpu.VMEM((2 * GROWS, D), f32),
            pltpu.VMEM((K + 1, RPW), f32),
            pltpu.SemaphoreType.DMA,
        ],
    )
    return run(center, outside, negflat, emb_v, emb_u)


def _tc_loss(B, K, scores):
    BL = 2048
    G = B // BL

    def body(s_ref, acc_ref):
        i = pl.program_id(0)
        x = s_ref[...]
        row = lax.broadcasted_iota(jnp.int32, (K + 1, BL), 0)
        y = jnp.clip(jnp.where(row == 0, x, -x), -10.0, 10.0)
        part = jnp.reshape(jnp.sum(jnp.log1p(jnp.exp(-y))), (1, 1))

        @pl.when(i == 0)
        def _():
            acc_ref[...] = jnp.zeros_like(acc_ref)

        acc_ref[...] += part

    out = pl.pallas_call(
        body,
        grid=(G,),
        in_specs=[pl.BlockSpec((K + 1, BL), lambda i: (0, i))],
        out_specs=pl.BlockSpec((1, 1), lambda i: (0, 0)),
        out_shape=jax.ShapeDtypeStruct((1, 1), jnp.float32),
        compiler_params=pltpu.CompilerParams(
            dimension_semantics=("arbitrary",)),
    )(scores)
    return out[0, 0]


def kernel(center, outside, negative, emb_v, emb_u):
    B, = center.shape
    K = negative.shape[1]
    D = emb_v.shape[1]
    scores = _sc_scores(B, K, D, center, outside, negative.reshape(-1),
                        emb_v, emb_u)
    return _tc_loss(B, K, scores)


# speedup vs baseline: 1.3169x; 1.3169x over previous
import functools

import jax
import jax.numpy as jnp
from jax import lax
from jax.experimental import pallas as pl
from jax.experimental.pallas import tpu as pltpu
from jax.experimental.pallas import tpu_sc as plsc

NC = 2
NS = 16
NW = NC * NS
PR = 32
GR = 16
RING = 8


def _sc_scores(B, K, D, center2d, outside2d, negk2d, emb_v, emb_u):
    RPW = B // NW
    NPH = RPW // PR
    CPP = PR * K // 128
    NCH = NPH * CPP
    VOCH = RPW // 128
    KPC = 128 // PR
    assert K % KPC == 0 and GR == 16

    mesh = plsc.VectorSubcoreMesh(core_axis_name="c", subcore_axis_name="s")
    f32 = jnp.float32

    def body(center_r, outside_r, negk_r, ev, eu, scores_hbm,
             idx_c, idx_o, idx_n, vbuf, obuf, nring, scores, gsem):
        wid = lax.axis_index("s") * NC + lax.axis_index("c")

        pltpu.sync_copy(center_r.at[pl.ds(wid * VOCH, VOCH)], idx_c)
        pltpu.sync_copy(outside_r.at[pl.ds(wid * VOCH, VOCH)], idx_o)
        pltpu.sync_copy(negk_r.at[pl.ds(wid * NCH, NCH)], idx_n)

        def n_pair(cg):
            slot = cg % RING
            return (eu, idx_n.at[cg], nring.at[pl.ds(slot * 128, 128)])

        def vo_pairs(p):
            q = p % 2
            return [(ev, idx_c.at[p], vbuf.at[pl.ds(q * 128, 128)]),
                    (eu, idx_o.at[p], obuf.at[pl.ds(q * 128, 128)])]

        def fire(t):
            tab, idx, dst = t
            pltpu.async_copy(tab.at[idx], dst, gsem)

        def drain(t):
            tab, idx, dst = t
            pltpu.make_async_copy(tab.at[idx], dst, gsem).wait()

        iota = lax.iota(jnp.int32, 16)

        def compute(np_, gp):
            q = ((np_ // 4) % 2) * 128
            rbase = (np_ % 4) * PR + gp * GR
            vrows = q + rbase + iota
            orows = vrows

            def jstep(j, carry):
                col = jnp.bitwise_and(iota + j, D - 1)
                vcol = plsc.load_gather(vbuf, [vrows, col])
                ocol = plsc.load_gather(obuf, [orows, col])
                out = [carry[0] + vcol * ocol]
                for k in range(K):
                    slot = (np_ * CPP + k // KPC) % RING
                    nbase = slot * 128 + (k % KPC) * PR + gp * GR
                    ncol = plsc.load_gather(nring, [nbase + iota, col])
                    out.append(carry[1 + k] + ncol * vcol)
                return tuple(out)

            zero = jnp.zeros((16,), f32)
            accs = lax.fori_loop(0, D, jstep, (zero,) * (K + 1))
            for k in range(K + 1):
                scores[k, pl.ds(np_ * PR + gp * GR, GR)] = accs[k]

        for t in vo_pairs(0):
            fire(t)
        for cg in range(RING):
            fire(n_pair(cg))
        fired = RING

        for np_ in range(NPH):
            if np_ % 4 == 0:
                for t in vo_pairs(np_ // 4):
                    drain(t)
            if np_ % 4 == 1 and np_ // 4 + 1 < VOCH:
                for t in vo_pairs(np_ // 4 + 1):
                    fire(t)
            base = np_ * CPP
            for c in range(3):
                drain(n_pair(base + c))
            compute(np_, 0)
            for _ in range(2):
                if fired < NCH:
                    fire(n_pair(fired))
                    fired += 1
            for c in range(3, CPP):
                drain(n_pair(base + c))
            compute(np_, 1)
            for _ in range(3):
                if fired < NCH:
                    fire(n_pair(fired))
                    fired += 1

        pltpu.sync_copy(scores, scores_hbm.at[:, pl.ds(wid * RPW, RPW)])

    run = pl.kernel(
        body,
        out_type=jax.ShapeDtypeStruct((K + 1, B), f32),
        mesh=mesh,
        compiler_params=pltpu.CompilerParams(
            use_tc_tiling_on_sc=False, needs_layout_passes=False),
        scratch_types=[
            pltpu.VMEM((VOCH, 128), jnp.int32),
            pltpu.VMEM((VOCH, 128), jnp.int32),
            pltpu.VMEM((NCH, 128), jnp.int32),
            pltpu.VMEM((2 * 128, D), f32),
            pltpu.VMEM((2 * 128, D), f32),
            pltpu.VMEM((RING * 128, D), f32),
            pltpu.VMEM((K + 1, RPW), f32),
            pltpu.SemaphoreType.DMA,
        ],
    )
    return run(center2d, outside2d, negk2d, emb_v, emb_u)


def _tc_loss(B, K, scores):
    BL = 2048
    G = B // BL

    def body(s_ref, acc_ref):
        i = pl.program_id(0)
        x = s_ref[...]
        row = lax.broadcasted_iota(jnp.int32, (K + 1, BL), 0)
        y = jnp.clip(jnp.where(row == 0, x, -x), -10.0, 10.0)
        part = jnp.reshape(jnp.sum(jnp.log1p(jnp.exp(-y))), (1, 1))

        @pl.when(i == 0)
        def _():
            acc_ref[...] = jnp.zeros_like(acc_ref)

        acc_ref[...] += part

    out = pl.pallas_call(
        body,
        grid=(G,),
        in_specs=[pl.BlockSpec((K + 1, BL), lambda i: (0, i))],
        out_specs=pl.BlockSpec((1, 1), lambda i: (0, 0)),
        out_shape=jax.ShapeDtypeStruct((1, 1), jnp.float32),
        compiler_params=pltpu.CompilerParams(
            dimension_semantics=("arbitrary",)),
    )(scores)
    return out[0, 0]


def kernel(center, outside, negative, emb_v, emb_u):
    B, = center.shape
    K = negative.shape[1]
    D = emb_v.shape[1]
    RPW = B // NW
    NPH = RPW // PR
    center2d = center.reshape(B // 128, 128)
    outside2d = outside.reshape(B // 128, 128)
    negk2d = (negative.reshape(NW, NPH, PR, K)
              .transpose(0, 1, 3, 2)
              .reshape(B * K // 128, 128))
    scores = _sc_scores(B, K, D, center2d, outside2d, negk2d,
                        emb_v, emb_u)
    return _tc_loss(B, K, scores)
